# baseline (speedup 1.0000x reference)
import functools
import jax
import jax.numpy as jnp
from jax import lax
from jax.experimental import pallas as pl
from jax.experimental.pallas import tpu as pltpu
from jax.experimental.pallas import tpu_sc as plsc

DIM = 64
SEQ = 200
BATCH = 1024
VOCAB = 1000000
NROWS = BATCH * SEQ
NW = 32
CHUNK = 128
NCHUNKS = NROWS // CHUNK
CH_PW = NCHUNKS // NW
SCALE = 8.0
NTILES = VOCAB // CHUNK
T_PW = NTILES // NW
T_XTRA = NTILES - T_PW * NW

_mesh = plsc.VectorSubcoreMesh(core_axis_name="c", subcore_axis_name="s")


@functools.partial(
    pl.kernel,
    mesh=_mesh,
    compiler_params=pltpu.CompilerParams(
        use_tc_tiling_on_sc=True, needs_layout_passes=False),
    out_type=jax.ShapeDtypeStruct((VOCAB // 2, 2 * DIM), jnp.float32),
    scratch_types=[
        pltpu.VMEM((DIM, CHUNK), jnp.float32),
        pltpu.VMEM((DIM, CHUNK), jnp.float32),
        pltpu.VMEM((DIM, CHUNK), jnp.float32),
        pltpu.VMEM((DIM, CHUNK), jnp.float32),
        pltpu.SemaphoreType.DMA,
        pltpu.SemaphoreType.DMA,
        pltpu.SemaphoreType.DMA,
        pltpu.SemaphoreType.DMA,
    ],
)
def _tr_kernel(tabt_hbm, tail_hbm, wt2_hbm, b0, b1, o0, o1, s0, s1, w0, w1):
    wid = lax.axis_index("s") * 2 + lax.axis_index("c")
    nt = T_PW + jnp.where(wid < T_XTRA, 1, 0)
    t0 = wid * T_PW + jnp.minimum(wid, T_XTRA)
    rows16 = lax.iota(jnp.int32, 16)

    def tile_base(t):
        return pl.multiple_of((t0 + t) * CHUNK, CHUNK)

    pltpu.async_copy(tabt_hbm.at[:, pl.ds(tile_base(0), CHUNK)], b0, s0)

    @pl.when(nt > 1)
    def _():
        pltpu.async_copy(tabt_hbm.at[:, pl.ds(tile_base(1), CHUNK)], b1, s1)

    def pair(c2, carry):
        for k, (buf, obuf, sem, wsem) in enumerate(
                ((b0, o0, s0, w0), (b1, o1, s1, w1))):
            c = c2 * 2 + k

            @pl.when(c < nt)
            def _process():
                pltpu.make_async_copy(
                    tabt_hbm.at[:, pl.ds(0, CHUNK)], buf, sem).wait()

                @pl.when(c >= 2)
                def _():
                    pltpu.make_async_copy(
                        obuf, wt2_hbm.at[pl.ds(0, DIM)], wsem).wait()

                rowvecs = [rows16 + d0 for d0 in range(0, DIM, 16)]
                for p in range(DIM):
                    for j in range(2 * DIM // 16):
                        par = j // (DIM // 16)
                        cols = jnp.broadcast_to(
                            jnp.int32(2 * p + par), (16,))
                        obuf[p, pl.ds(j * 16, 16)] = plsc.load_gather(
                            buf, [rowvecs[j % (DIM // 16)], cols])
                pltpu.async_copy(
                    obuf,
                    wt2_hbm.at[pl.ds(
                        pl.multiple_of(tile_base(c) // 2, DIM), DIM)],
                    wsem)

                @pl.when(c + 2 < nt)
                def _():
                    pltpu.async_copy(
                        tabt_hbm.at[:, pl.ds(tile_base(c + 2), CHUNK)],
                        buf, sem)

        return carry

    lax.fori_loop(0, (T_PW + 2) // 2, pair, 0)

    @pl.when(wid == 0)
    def _tail():
        pltpu.sync_copy(
            tail_hbm, wt2_hbm.at[pl.ds(NTILES * CHUNK // 2, DIM // 2)])

    @pl.when(nt >= 2)
    def _():
        pltpu.make_async_copy(o0, wt2_hbm.at[pl.ds(0, DIM)], w0).wait()
        pltpu.make_async_copy(o1, wt2_hbm.at[pl.ds(0, DIM)], w1).wait()


@functools.partial(
    pl.kernel,
    mesh=_mesh,
    out_type=jax.ShapeDtypeStruct((NROWS, DIM), jnp.float32),
    scratch_types=[
        pltpu.VMEM((CH_PW, CHUNK), jnp.int32),
        pltpu.VMEM((CH_PW, CHUNK), jnp.int32),
        pltpu.VMEM((2 * SEQ, DIM), jnp.float32),
        pltpu.VMEM((CHUNK, 2 * DIM), jnp.float32),
        pltpu.VMEM((CHUNK, 2 * DIM), jnp.float32),
        pltpu.VMEM((CHUNK, DIM), jnp.float32),
        pltpu.SemaphoreType.DMA,
        pltpu.SemaphoreType.DMA,
    ],
)
def _emb_kernel(idx_hbm, off_hbm, pe_hbm, tab_hbm, out_hbm, idx_v, off_v,
                pe_v, buf0, buf1, obuf, s0, s1):
    wid = lax.axis_index("s") * 2 + lax.axis_index("c")

    pltpu.sync_copy(idx_hbm.at[wid], idx_v)
    pltpu.sync_copy(off_hbm.at[wid], off_v)
    pltpu.sync_copy(pe_hbm, pe_v)

    pltpu.async_copy(tab_hbm.at[idx_v.at[0]], buf0, s0)
    pltpu.async_copy(tab_hbm.at[idx_v.at[1]], buf1, s1)

    def pair(c2, carry):
        for k, (buf, sem) in enumerate(((buf0, s0), (buf1, s1))):
            c = c2 * 2 + k
            pltpu.make_async_copy(tab_hbm.at[idx_v.at[0]], buf, sem).wait()
            r0 = (wid * CH_PW + c) * CHUNK
            p0 = lax.rem(r0, SEQ)

            def body(g, _):
                off16 = off_v[c, pl.ds(g * 16, 16)]
                for i in range(16):
                    off = off16[i]
                    r = g * 16 + i
                    pr = p0 + r
                    for j in range(DIM // 16):
                        obuf[r, pl.ds(j * 16, 16)] = (
                            buf[r, pl.ds(off + j * 16, 16)] * SCALE
                            + pe_v[pr, pl.ds(j * 16, 16)]
                        )
                return 0

            lax.fori_loop(0, CHUNK // 16, body, 0)
            pltpu.sync_copy(obuf, out_hbm.at[pl.ds(r0, CHUNK)])

            @pl.when(c2 < CH_PW // 2 - 1)
            def _():
                pltpu.async_copy(tab_hbm.at[idx_v.at[c + 2]], buf, sem)

        return carry

    lax.fori_loop(0, CH_PW // 2, pair, 0)


def kernel(x, embed_weight, pe):
    x3 = x.reshape(NW, CH_PW, CHUNK).astype(jnp.int32)
    idx = x3 >> 1
    off = (x3 & 1) * DIM
    tail = embed_weight[NTILES * CHUNK:].reshape(DIM // 2, 2 * DIM)
    wt2 = _tr_kernel(embed_weight.T, tail)
    pe1 = pe[0, :SEQ].astype(jnp.float32)
    pe2 = jnp.concatenate([pe1, pe1], axis=0)
    out = _emb_kernel(idx, off, pe2, wt2)
    return out.reshape(BATCH, SEQ, DIM)


# speedup vs baseline: 2.2133x; 2.2133x over previous
import functools
import jax
import jax.numpy as jnp
from jax import lax
from jax.experimental import pallas as pl
from jax.experimental.pallas import tpu as pltpu
from jax.experimental.pallas import tpu_sc as plsc

DIM = 64
SEQ = 200
BATCH = 1024
NW = 32
B_PW = BATCH // NW
CHUNK = SEQ // 2
SCALE = 8.0

_mesh = plsc.VectorSubcoreMesh(core_axis_name="c", subcore_axis_name="s")


@functools.partial(
    pl.kernel,
    mesh=_mesh,
    compiler_params=pltpu.CompilerParams(use_tc_tiling_on_sc=False),
    out_type=jax.ShapeDtypeStruct((BATCH, SEQ, DIM), jnp.float32),
    scratch_types=[
        pltpu.VMEM((2 * B_PW, CHUNK), jnp.int32),
        pltpu.VMEM((SEQ, DIM), jnp.float32),
        pltpu.VMEM((CHUNK, DIM), jnp.float32),
        pltpu.VMEM((CHUNK, DIM), jnp.float32),
        pltpu.SemaphoreType.DMA,
        pltpu.SemaphoreType.DMA,
    ],
)
def _emb_kernel(idx_hbm, pe_hbm, tab_hbm, out_hbm, idx_v, pe_v, buf0, buf1,
                s0, s1):
    wid = lax.axis_index("s") * 2 + lax.axis_index("c")
    b0 = wid * B_PW
    cbase = wid * 2 * B_PW

    pltpu.sync_copy(idx_hbm.at[pl.ds(cbase, 2 * B_PW)], idx_v)
    pltpu.sync_copy(pe_hbm, pe_v)

    pltpu.async_copy(tab_hbm.at[idx_v.at[0]], buf0, s0)
    pltpu.async_copy(tab_hbm.at[idx_v.at[1]], buf1, s1)

    def seq_step(bb, carry):
        for k, (buf, sem) in enumerate(((buf0, s0), (buf1, s1))):
            pltpu.make_async_copy(tab_hbm.at[idx_v.at[0]], buf, sem).wait()

            def body(r, _):
                pr = k * CHUNK + r
                for j in range(DIM // 16):
                    sl = pl.ds(j * 16, 16)
                    buf[r, sl] = buf[r, sl] * SCALE + pe_v[pr, sl]
                return 0

            lax.fori_loop(0, CHUNK, body, 0)
            pltpu.sync_copy(
                buf, out_hbm.at[b0 + bb, pl.ds(k * CHUNK, CHUNK)])

            @pl.when(bb < B_PW - 1)
            def _():
                pltpu.async_copy(
                    tab_hbm.at[idx_v.at[2 * bb + 2 + k]], buf, sem)

        return carry

    lax.fori_loop(0, B_PW, seq_step, 0)


def kernel(x, embed_weight, pe):
    idx = x.reshape(BATCH * SEQ // CHUNK, CHUNK).astype(jnp.int32)
    pe2 = pe[0, :SEQ].astype(jnp.float32)
    return _emb_kernel(idx, pe2, embed_weight)
